# Initial kernel scaffold; baseline (speedup 1.0000x reference)
import jax
import jax.numpy as jnp
from jax.experimental import pallas as pl


def kernel(z, e):
    raise NotImplementedError("write your pallas kernel here")


# speedup vs baseline: 1.1231x; 1.1231x over previous
import functools

import jax
import jax.numpy as jnp
from jax.experimental import pallas as pl
from jax.experimental.pallas import tpu as pltpu
from jax.experimental.pallas import tpu_sc as plsc

N, K, D = 18432, 1024, 64
BLK = 512


def _vq_tc_kernel(z_ref, e_ref, zz_ref, ee_ref, idx_ref, loss_ref):
    i = pl.program_id(0)
    z = z_ref[...]
    e = e_ref[...]
    ze = jax.lax.dot_general(z, e, (((1,), (1,)), ((), ())),
                             preferred_element_type=jnp.float32)
    d2 = (zz_ref[...] - 2.0 * ze) + ee_ref[...]
    dist = jnp.sqrt(jnp.maximum(d2, 0.0))
    m = jnp.min(dist, axis=1, keepdims=True)
    iota = jax.lax.broadcasted_iota(jnp.int32, (BLK, K), 1)
    idx = jnp.min(jnp.where(dist == m, iota, K), axis=1, keepdims=True)
    idx_ref[...] = idx
    m2sum = jnp.sum(m * m).reshape(1, 1)

    @pl.when(i == 0)
    def _():
        loss_ref[...] = m2sum

    @pl.when(i > 0)
    def _():
        loss_ref[...] += m2sum


def _argmin_distances(z, e, zz, ee):
    grid = N // BLK
    return pl.pallas_call(
        _vq_tc_kernel,
        grid=(grid,),
        in_specs=[
            pl.BlockSpec((BLK, D), lambda i: (i, 0)),
            pl.BlockSpec((K, D), lambda i: (0, 0)),
            pl.BlockSpec((BLK, 1), lambda i: (i, 0)),
            pl.BlockSpec((1, K), lambda i: (0, 0)),
        ],
        out_specs=[
            pl.BlockSpec((BLK, 1), lambda i: (i, 0)),
            pl.BlockSpec((1, 1), lambda i: (0, 0)),
        ],
        out_shape=[
            jax.ShapeDtypeStruct((N, 1), jnp.int32),
            jax.ShapeDtypeStruct((1, 1), jnp.float32),
        ],
    )(z, e, zz, ee)


def _gather_codebook(e, indices):
    num_indices = indices.shape[0]
    w = 128
    mesh = plsc.VectorSubcoreMesh(core_axis_name="core",
                                  subcore_axis_name="subcore")
    idx2 = indices.reshape(1, num_indices)
    epad = jnp.concatenate([e, jnp.zeros((K, D), e.dtype)], axis=1)

    @functools.partial(
        pl.kernel,
        out_type=jax.ShapeDtypeStruct((num_indices, 2 * D), e.dtype),
        mesh=mesh)
    def k(e_hbm, i_hbm, o_hbm):
        def body(i_vmem, o_vmem):
            pltpu.sync_copy(e_hbm.at[i_vmem.at[0]], o_vmem)

        pltpu.emit_pipeline(
            body,
            grid=(num_indices // w,),
            in_specs=[pl.BlockSpec((1, w), index_map=lambda i: (0, i))],
            out_specs=[pl.BlockSpec((w, 2 * D), index_map=lambda i: (i, 0))],
            core_axis_name=("core", "subcore"),
            dimension_semantics=(pltpu.PARALLEL,),
        )(i_hbm, o_hbm)

    return k(epad, idx2)[:, :D]


def kernel(z, e):
    zz = jnp.sum(z * z, axis=1, keepdims=True)
    ee = jnp.sum(e * e, axis=1)[None, :]
    idx2, loss_sum = _argmin_distances(z, e, zz, ee)
    min_indices = idx2.reshape(N)
    zq = _gather_codebook(e, min_indices)
    commit_loss = loss_sum[0, 0] / (N * D)
    return zq, min_indices, commit_loss


# speedup vs baseline: 1.4039x; 1.2500x over previous
import functools

import jax
import jax.numpy as jnp
from jax.experimental import pallas as pl
from jax.experimental.pallas import tpu as pltpu
from jax.experimental.pallas import tpu_sc as plsc

N, K, D = 18432, 1024, 64
BLK = 512


def _vq_tc_kernel(e2_ref, zt_ref, zz_ref, ee_ref, idx_ref, loss_ref):
    i = pl.program_id(0)
    e2 = e2_ref[...]
    zt = zt_ref[...]
    ze2 = jax.lax.dot_general(e2, zt, (((1,), (0,)), ((), ())),
                              preferred_element_type=jnp.float32)
    d2 = (zz_ref[...] + ze2) + ee_ref[...]
    c = jnp.maximum(d2, 0.0)
    m2 = c
    half = K // 2
    while half >= 1:
        m2 = jnp.minimum(m2[:half], m2[half:])
        half //= 2
    s = jnp.sqrt(m2)
    p_bits = jax.lax.bitcast_convert_type(s * s, jnp.int32)
    t = jnp.full_like(m2, -jnp.inf)
    for kk in range(-3, 4):
        xk = jax.lax.bitcast_convert_type(p_bits + kk, jnp.float32)
        t = jnp.where(jnp.sqrt(xk) == s, xk, t)
    iota = jax.lax.broadcasted_iota(
        jnp.int32, (K, BLK), 0).astype(jnp.float32)
    ix = jnp.where(c <= t, iota, float(K))
    half = K // 2
    while half >= 1:
        ix = jnp.minimum(ix[:half], ix[half:])
        half //= 2
    idx_ref[...] = ix.astype(jnp.int32).reshape(1, 1, BLK)
    m2sum = jnp.sum(m2).reshape(1, 1)

    @pl.when(i == 0)
    def _():
        loss_ref[...] = m2sum

    @pl.when(i > 0)
    def _():
        loss_ref[...] += m2sum


def _argmin_distances(zt, e2, zz, ee):
    grid = N // BLK
    return pl.pallas_call(
        _vq_tc_kernel,
        grid=(grid,),
        in_specs=[
            pl.BlockSpec((K, D), lambda i: (0, 0)),
            pl.BlockSpec((D, BLK), lambda i: (0, i)),
            pl.BlockSpec((1, BLK), lambda i: (0, i)),
            pl.BlockSpec((K, 1), lambda i: (0, 0)),
        ],
        out_specs=[
            pl.BlockSpec((1, 1, BLK), lambda i: (i, 0, 0)),
            pl.BlockSpec((1, 1), lambda i: (0, 0)),
        ],
        out_shape=[
            jax.ShapeDtypeStruct((grid, 1, BLK), jnp.int32),
            jax.ShapeDtypeStruct((1, 1), jnp.float32),
        ],
    )(e2, zt, zz, ee)


def _gather_codebook(e, indices):
    num_indices = indices.shape[0]
    w = 128
    mesh = plsc.VectorSubcoreMesh(core_axis_name="core",
                                  subcore_axis_name="subcore")
    idx2 = indices.reshape(1, num_indices)
    epad = jnp.concatenate([e, jnp.zeros((K, D), e.dtype)], axis=1)

    @functools.partial(
        pl.kernel,
        out_type=jax.ShapeDtypeStruct((num_indices, 2 * D), e.dtype),
        mesh=mesh)
    def k(e_hbm, i_hbm, o_hbm):
        def body(i_vmem, o_vmem):
            pltpu.sync_copy(e_hbm.at[i_vmem.at[0]], o_vmem)

        pltpu.emit_pipeline(
            body,
            grid=(num_indices // w,),
            in_specs=[pl.BlockSpec((1, w), index_map=lambda i: (0, i))],
            out_specs=[pl.BlockSpec((w, 2 * D), index_map=lambda i: (i, 0))],
            core_axis_name=("core", "subcore"),
            dimension_semantics=(pltpu.PARALLEL,),
        )(i_hbm, o_hbm)

    return k(epad, idx2)[:, :D]


def kernel(z, e):
    zz = jnp.sum(z * z, axis=1)[None, :]
    ee = jnp.sum(e * e, axis=1, keepdims=True)
    zt = z.T
    e2 = -2.0 * e
    idx3, loss_sum = _argmin_distances(zt, e2, zz, ee)
    min_indices = idx3.reshape(N)
    zq = _gather_codebook(e, min_indices)
    commit_loss = loss_sum[0, 0] / (N * D)
    return zq, min_indices, commit_loss
